# baseline (speedup 1.0000x reference)
import functools

import jax
import jax.numpy as jnp
from jax import lax
from jax.experimental import pallas as pl
from jax.experimental.pallas import tpu as pltpu
from jax.experimental.pallas import tpu_sc as plsc

N = 4608
K = 8192
D = 256
NB = 256
NUM_SC_WORKERS = 32
BPW = N // NUM_SC_WORKERS
HALF = BPW // 2


def _argmin_level0_body(x_ref, w_ref, idx_ref):
    _argmin_common(x_ref[...], w_ref[...], idx_ref)


def _argmin_level1_body(x_ref, q_ref, w_ref, idx_ref):
    _argmin_common(x_ref[...] - q_ref[...], w_ref[...], idx_ref)


def _argmin_common(x, w, idx_ref):
    x2 = jnp.sum(x * x, axis=1, keepdims=True)
    w2 = jnp.sum(w * w, axis=1)[None, :]
    dot = lax.dot_general(x, w, (((1,), (1,)), ((), ())))
    d2 = (x2 + w2) - 2.0 * dot
    dist = jnp.sqrt(jnp.maximum(d2, 0.0))
    m = jnp.min(dist, axis=1, keepdims=True)
    iota = lax.broadcasted_iota(jnp.int32, dist.shape, 1)
    idx = jnp.min(jnp.where(dist == m, iota, K), axis=1)
    idx_ref[...] = jnp.broadcast_to(idx[:, None], (idx_ref.shape[0], 128))


_x_spec = pl.BlockSpec((NB, D), lambda i: (i, 0))
_w_spec = pl.BlockSpec((K, D), lambda i: (0, 0))
_idx_spec = pl.BlockSpec((NB, 128), lambda i: (i, 0))
_idx_shape = jax.ShapeDtypeStruct((N, 128), jnp.int32)

_argmin0 = pl.pallas_call(
    _argmin_level0_body,
    grid=(N // NB,),
    in_specs=[_x_spec, _w_spec],
    out_specs=_idx_spec,
    out_shape=_idx_shape,
)

_argmin1 = pl.pallas_call(
    _argmin_level1_body,
    grid=(N // NB,),
    in_specs=[_x_spec, _x_spec, _w_spec],
    out_specs=_idx_spec,
    out_shape=_idx_shape,
)


@functools.cache
def _get_gather_sc():
    @functools.partial(
        pl.kernel,
        mesh=plsc.VectorSubcoreMesh(core_axis_name="c", subcore_axis_name="s"),
        out_type=jax.ShapeDtypeStruct((N, D), jnp.float32),
        scratch_types=[
            pltpu.VMEM((BPW,), jnp.int32),
            pltpu.VMEM((BPW, D), jnp.float32),
            pltpu.SemaphoreType.DMA,
        ],
    )
    def _gather_sc(table_hbm, idx_hbm, out_hbm, idx_v, rows_v, sem):
        wid = lax.axis_index("s") * 2 + lax.axis_index("c")
        base = wid * BPW
        pltpu.sync_copy(idx_hbm.at[pl.ds(base, BPW)], idx_v)
        cp0 = pltpu.async_copy(
            table_hbm.at[idx_v.at[pl.ds(0, HALF)]], rows_v.at[pl.ds(0, HALF)], sem)
        cp1 = pltpu.async_copy(
            table_hbm.at[idx_v.at[pl.ds(HALF, HALF)]], rows_v.at[pl.ds(HALF, HALF)], sem)
        cp0.wait()
        cp1.wait()
        pltpu.sync_copy(rows_v, out_hbm.at[pl.ds(base, BPW)])

    return _gather_sc


def _final_body(x_ref, q0_ref, q1_ref, out_ref, s0_ref, s1_ref):
    i = pl.program_id(0)
    x = x_ref[...]
    q0 = q0_ref[...]
    q1 = q1_ref[...]
    code_sum = q0 + q1
    out_ref[...] = x + (code_sum - x)
    d0 = q0 - x
    d1 = q1 - (x - q0)

    @pl.when(i == 0)
    def _():
        s0_ref[0, 0] = 0.0
        s1_ref[0, 0] = 0.0

    s0_ref[0, 0] += jnp.sum(d0 * d0)
    s1_ref[0, 0] += jnp.sum(d1 * d1)


_final = pl.pallas_call(
    _final_body,
    grid=(N // NB,),
    in_specs=[_x_spec, _x_spec, _x_spec],
    out_specs=[
        pl.BlockSpec((NB, D), lambda i: (i, 0)),
        pl.BlockSpec(memory_space=pltpu.SMEM, block_shape=(1, 1), index_map=lambda i: (0, 0)),
        pl.BlockSpec(memory_space=pltpu.SMEM, block_shape=(1, 1), index_map=lambda i: (0, 0)),
    ],
    out_shape=[
        jax.ShapeDtypeStruct((N, D), jnp.float32),
        jax.ShapeDtypeStruct((1, 1), jnp.float32),
        jax.ShapeDtypeStruct((1, 1), jnp.float32),
    ],
)


def kernel(latent, W0, W1):
    gather_sc = _get_gather_sc()
    idx0 = _argmin0(latent, W0)[:, 0]
    q0 = gather_sc(W0, idx0)
    idx1 = _argmin1(latent, q0, W1)[:, 0]
    q1 = gather_sc(W1, idx1)
    out, s0, s1 = _final(latent, q0, q1)
    nd = jnp.float32(N * D)
    l0 = s0[0, 0] / nd
    l1 = s1[0, 0] / nd
    loss = l0 + 0.25 * l0 + l1 + 0.25 * l1
    return (loss, out)
